# baseline (speedup 1.0000x reference)
import functools

import jax
import jax.numpy as jnp
from jax import lax
from jax.experimental import pallas as pl
from jax.experimental.pallas import tpu as pltpu
from jax.experimental.pallas import tpu_sc as plsc

F32 = jnp.float32
_NC = 2
_NS = 16
_NW = _NC * _NS
_L = 16
_H = 64
_D = 128
_K = 80


def _mesh():
    return plsc.VectorSubcoreMesh(core_axis_name="c", subcore_axis_name="s")


def _sc_gather(E, N):
    EPW = E // _NW
    NBLK = EPW // _K
    PAIRS = (NBLK - 1) // 2

    @functools.partial(
        pl.kernel,
        out_type=[
            jax.ShapeDtypeStruct((E, _H), F32),
            jax.ShapeDtypeStruct((_NW * _D,), F32),
        ],
        mesh=_mesh(),
        scratch_types=[
            pltpu.VMEM((EPW,), jnp.int32),
            pltpu.VMEM((EPW,), jnp.int32),
            pltpu.VMEM((_K, _D), F32),
            pltpu.VMEM((_K, _D), F32),
            pltpu.VMEM((_K, _D), F32),
            pltpu.VMEM((_K, _D), F32),
            pltpu.VMEM((_K, _H), F32),
            pltpu.VMEM((_K, _H), F32),
            pltpu.VMEM((_D,), F32),
            pltpu.SemaphoreType.DMA,
            pltpu.SemaphoreType.DMA,
            pltpu.SemaphoreType.DMA,
            pltpu.SemaphoreType.DMA,
        ],
    )
    def k(ab, srcv, dstv, h_out, st_out,
          isv, idv, ra0, ra1, rb0, rb1, hb0, hb1, sb, sA0, sA1, sO0, sO1):
        cid = lax.axis_index("c")
        sid = lax.axis_index("s")
        wid = sid * _NC + cid
        base = wid * EPW
        zero = jnp.zeros((_L,), F32)

        pltpu.sync_copy(srcv.at[pl.ds(base, EPW)], isv)
        pltpu.sync_copy(dstv.at[pl.ds(base, EPW)], idv)

        def issue(b, ra, rb, sem):
            pltpu.async_copy(ab.at[idv.at[pl.ds(b * _K, _K)]], ra, sem)
            pltpu.async_copy(ab.at[isv.at[pl.ds(b * _K, _K)]], rb, sem)

        def drain_g(ra, rb, sem):
            pltpu.make_async_copy(ab.at[pl.ds(0, _K)], ra, sem).wait()
            pltpu.make_async_copy(ab.at[pl.ds(0, _K)], rb, sem).wait()

        def out(b, hb, sem):
            pltpu.async_copy(hb, h_out.at[pl.ds(base + b * _K, _K)], sem)

        def drain_o(hb, sem):
            pltpu.make_async_copy(hb, h_out.at[pl.ds(base, _K)], sem).wait()

        def compute(ra, rb, hb, carry):
            def row(r, c8):
                acc = list(c8)
                for c in range(4):
                    h = (ra[r, pl.ds(c * _L, _L)]
                         + rb[r, pl.ds(_H + c * _L, _L)])
                    hb[r, pl.ds(c * _L, _L)] = h
                    acc[c] = acc[c] + h
                    acc[4 + c] = acc[4 + c] + h * h
                return tuple(acc)

            return lax.fori_loop(0, _K, row, carry, unroll=8)

        issue(0, ra0, rb0, sA0)
        out(0, hb0, sO0)
        out(1, hb1, sO1)

        def pair(t, carry):
            a = 2 * t
            issue(a + 1, ra1, rb1, sA1)
            drain_g(ra0, rb0, sA0)
            drain_o(hb0, sO0)
            carry = compute(ra0, rb0, hb0, carry)
            out(a, hb0, sO0)
            issue(a + 2, ra0, rb0, sA0)
            drain_g(ra1, rb1, sA1)
            drain_o(hb1, sO1)
            carry = compute(ra1, rb1, hb1, carry)
            out(a + 1, hb1, sO1)
            return carry

        carry = lax.fori_loop(0, PAIRS, pair, (zero,) * 8)
        drain_g(ra0, rb0, sA0)
        drain_o(hb0, sO0)
        carry = compute(ra0, rb0, hb0, carry)
        out(NBLK - 1, hb0, sO0)
        drain_o(hb0, sO0)
        drain_o(hb1, sO1)

        for c in range(4):
            sb[pl.ds(c * _L, _L)] = carry[c]
            sb[pl.ds(_H + c * _L, _L)] = carry[4 + c]
        pltpu.sync_copy(sb, st_out.at[pl.ds(wid * _D, _D)])

    return k


def _sc_scatter(E, N):
    EPW = E // _NW
    NBLK = EPW // _K
    PAIRS = (NBLK - 1) // 2
    CH = 40
    NCHUNK = N // CH
    TPW = (NCHUNK + _NS - 1) // _NS

    @functools.partial(
        pl.kernel,
        out_type=jax.ShapeDtypeStruct((_NC, N, _D), F32),
        mesh=_mesh(),
        scratch_types=[
            pltpu.VMEM_SHARED((N, _D), F32),
            pltpu.VMEM((NBLK, _K), jnp.int32),
            pltpu.VMEM((_K, _D), F32),
            pltpu.VMEM((_K, _D), F32),
            pltpu.VMEM((CH, _D), F32),
            pltpu.VMEM((_D,), F32),
            pltpu.SemaphoreType.DMA,
            pltpu.SemaphoreType.DMA,
        ],
    )
    def k(h3, dst3, st_hbm, part,
          acc, idx3, pb0, pb1, zbuf, stv, sS0, sS1):
        cid = lax.axis_index("c")
        sid = lax.axis_index("s")
        wid = sid * _NC + cid
        base = wid * EPW
        pltpu.sync_copy(st_hbm, stv)
        pltpu.sync_copy(dst3.at[wid], idx3)

        zv = jnp.zeros((_L,), F32)

        def zrow(r, _):
            for c in range(_D // _L):
                zbuf[r, pl.ds(c * _L, _L)] = zv
            return 0

        lax.fori_loop(0, CH, zrow, 0)
        for t in range(TPW):
            c = sid * TPW + t

            @pl.when(c < NCHUNK)
            def _():
                pltpu.sync_copy(zbuf, acc.at[pl.ds(c * CH, CH)])

        def pzrow(r, _):
            for c in range(_D // _L):
                pb0[r, pl.ds(c * _L, _L)] = zv
                pb1[r, pl.ds(c * _L, _L)] = zv
            return 0

        lax.fori_loop(0, _K, pzrow, 0)
        plsc.subcore_barrier()

        svec = [stv[pl.ds(c * _L, _L)] for c in range(4)]
        tvec = [stv[pl.ds(_H + c * _L, _L)] for c in range(4)]

        def scat(b, pb, sem):
            pltpu.async_copy(pb, acc.at[idx3.at[b]], sem, add=True)

        def drain_s(pb, sem):
            pltpu.make_async_copy(pb, acc.at[idx3.at[0]], sem).wait()

        def block(b, pb, sem):
            drain_s(pb, sem)
            pltpu.sync_copy(h3.at[pl.ds(base + b * _K, _K)], pb)

            def row(r, _2):
                for c in range(4):
                    v = pb[r, pl.ds(c * _L, _L)]
                    pb[r, pl.ds(c * _L, _L)] = jnp.maximum(
                        v * svec[c] + tvec[c], 0.0)
                return 0

            lax.fori_loop(0, _K, row, 0, unroll=8)
            scat(b, pb, sem)

        scat(0, pb0, sS0)
        scat(1, pb1, sS1)

        def pair(t, _):
            a = 2 * t
            block(a, pb0, sS0)
            block(a + 1, pb1, sS1)
            return 0

        lax.fori_loop(0, PAIRS, pair, 0)
        block(NBLK - 1, pb0, sS0)
        drain_s(pb0, sS0)
        drain_s(pb1, sS1)
        plsc.subcore_barrier()

        for t in range(TPW):
            c = sid * TPW + t

            @pl.when(c < NCHUNK)
            def _():
                pltpu.sync_copy(acc.at[pl.ds(c * CH, CH)], zbuf)
                pltpu.sync_copy(zbuf, part.at[cid, pl.ds(c * CH, CH)])

    return k


def _sc_final(E, N):
    EPW = E // _NW
    NBLK = EPW // _K
    PAIRS = (NBLK - 1) // 2

    @functools.partial(
        pl.kernel,
        out_type=jax.ShapeDtypeStruct((E, _L), F32),
        mesh=_mesh(),
        scratch_types=[
            pltpu.VMEM((EPW,), jnp.int32),
            pltpu.VMEM((EPW,), jnp.int32),
            pltpu.VMEM((_K, _D), F32),
            pltpu.VMEM((_K, _D), F32),
            pltpu.VMEM((_K, _D), F32),
            pltpu.VMEM((_K, _D), F32),
            pltpu.VMEM((_K, _L), F32),
            pltpu.VMEM((_K, _L), F32),
            pltpu.VMEM((272,), F32),
            pltpu.SemaphoreType.DMA,
            pltpu.SemaphoreType.DMA,
            pltpu.SemaphoreType.DMA,
            pltpu.SemaphoreType.DMA,
        ],
    )
    def k(c_hbm, srcv, dstv, prm, o_hbm,
          isv, idv, rs0, rs1, rd0, rd1, ob0, ob1, pv, sA0, sA1, sO0, sO1):
        cid = lax.axis_index("c")
        sid = lax.axis_index("s")
        wid = sid * _NC + cid
        base = wid * EPW

        pltpu.sync_copy(prm, pv)
        pltpu.sync_copy(srcv.at[pl.ds(base, EPW)], isv)
        pltpu.sync_copy(dstv.at[pl.ds(base, EPW)], idv)
        bev = [pv[pl.ds(c * _L, _L)] for c in range(_D // _L)]
        wev = [pv[pl.ds(_D + c * _L, _L)] for c in range(_D // _L)]

        def issue(b, rs, rd, sem):
            pltpu.async_copy(c_hbm.at[isv.at[pl.ds(b * _K, _K)]], rs, sem)
            pltpu.async_copy(c_hbm.at[idv.at[pl.ds(b * _K, _K)]], rd, sem)

        def drain_g(rs, rd, sem):
            pltpu.make_async_copy(c_hbm.at[pl.ds(0, _K)], rs, sem).wait()
            pltpu.make_async_copy(c_hbm.at[pl.ds(0, _K)], rd, sem).wait()

        def out(b, ob, sem):
            pltpu.async_copy(ob, o_hbm.at[pl.ds(base + b * _K, _K)], sem)

        def drain_o(ob, sem):
            pltpu.make_async_copy(ob, o_hbm.at[pl.ds(base, _K)], sem).wait()

        def compute(rs, rd, ob):
            def row(r, _2):
                accs = []
                for c in range(_D // _L):
                    dv = rs[r, pl.ds(c * _L, _L)] - rd[r, pl.ds(c * _L, _L)]
                    dv = jnp.maximum(dv + bev[c], 0.0)
                    accs.append(dv * wev[c])
                while len(accs) > 1:
                    accs = [a + b for a, b in zip(accs[::2], accs[1::2])]
                ob[r, pl.ds(0, _L)] = accs[0]
                return 0

            lax.fori_loop(0, _K, row, 0, unroll=8)

        issue(0, rs0, rd0, sA0)
        out(0, ob0, sO0)
        out(1, ob1, sO1)

        def pair(t, _):
            a = 2 * t
            issue(a + 1, rs1, rd1, sA1)
            drain_g(rs0, rd0, sA0)
            drain_o(ob0, sO0)
            compute(rs0, rd0, ob0)
            out(a, ob0, sO0)
            issue(a + 2, rs0, rd0, sA0)
            drain_g(rs1, rd1, sA1)
            drain_o(ob1, sO1)
            compute(rs1, rd1, ob1)
            out(a + 1, ob1, sO1)
            return 0

        lax.fori_loop(0, PAIRS, pair, 0)
        drain_g(rs0, rd0, sA0)
        drain_o(ob0, sO0)
        compute(rs0, rd0, ob0)
        out(NBLK - 1, ob0, sO0)
        drain_o(ob0, sO0)
        drain_o(ob1, sO1)

    return k


def _tc_pre(x, W1):
    N, D = x.shape
    BN = 2000

    def body(x_ref, w_ref, ab_ref):
        w = w_ref[...]
        wa = w[:D, :]
        wb = w[D:, :]
        xb = x_ref[...]
        ab_ref[...] = jnp.concatenate(
            [jnp.dot(xb, wa - wb, preferred_element_type=F32),
             jnp.dot(xb, wb, preferred_element_type=F32)], axis=1)

    return pl.pallas_call(
        body,
        grid=(N // BN,),
        in_specs=[pl.BlockSpec((BN, D), lambda i: (i, 0)),
                  pl.BlockSpec((2 * D, _H), lambda i: (0, 0))],
        out_specs=pl.BlockSpec((BN, 2 * _H), lambda i: (i, 0)),
        out_shape=jax.ShapeDtypeStruct((N, 2 * _H), F32),
    )(x, W1)


def _tc_mid(h, st, W, pad_count=False):
    E = h.shape[0]
    BE = 2560
    G = E // BE
    OW = _D if pad_count else _H

    def body(h_ref, st_ref, w_ref, o_ref, ss_ref, sq_ref):
        i = pl.program_id(0)
        stb = st_ref[...]
        hb = h_ref[...].astype(F32)
        h1 = jnp.maximum(hb * stb[0:1, :] + stb[1:2, :], 0.0)
        o = jnp.dot(h1, w_ref[...], preferred_element_type=F32)
        if pad_count:
            cols = lax.broadcasted_iota(jnp.int32, (BE, _H), 1)
            cc = jnp.where(cols == 0, jnp.float32(1), jnp.float32(0))
            o_ref[...] = jnp.concatenate([o, cc], axis=1)
        else:
            o_ref[...] = o.astype(jnp.bfloat16)
        s = jnp.broadcast_to(jnp.sum(o, axis=0, keepdims=True), (8, _H))
        q = jnp.broadcast_to(jnp.sum(o * o, axis=0, keepdims=True), (8, _H))

        @pl.when(i == 0)
        def _():
            ss_ref[...] = s
            sq_ref[...] = q

        @pl.when(i > 0)
        def _():
            ss_ref[...] = ss_ref[...] + s
            sq_ref[...] = sq_ref[...] + q

    out, ss, sq = pl.pallas_call(
        body,
        grid=(G,),
        in_specs=[pl.BlockSpec((BE, _H), lambda i: (i, 0)),
                  pl.BlockSpec((2, _H), lambda i: (0, 0)),
                  pl.BlockSpec((_H, _H), lambda i: (0, 0))],
        out_specs=[pl.BlockSpec((BE, OW), lambda i: (i, 0)),
                   pl.BlockSpec((8, _H), lambda i: (0, 0)),
                   pl.BlockSpec((8, _H), lambda i: (0, 0))],
        out_shape=[jax.ShapeDtypeStruct((E, OW),
                                        F32 if pad_count else jnp.bfloat16),
                   jax.ShapeDtypeStruct((8, _H), F32),
                   jax.ShapeDtypeStruct((8, _H), F32)],
    )(h, st, W)
    return out, ss[0], sq[0]


def _tc_combine(parts, x, We1):
    N, D = x.shape
    BN = 2000

    def body(p_ref, x_ref, w_ref, c_ref):
        p = p_ref[...]
        ps = p[0] + p[1]
        cnt = jnp.maximum(ps[:, _H:_H + 1], 1.0)
        agg = jnp.maximum(ps[:, :_H] / cnt, 0.0)
        xb = jnp.maximum(x_ref[...], 0.0)
        w = w_ref[...]
        c_ref[...] = (jnp.dot(agg, w[:_H, :], preferred_element_type=F32)
                      + jnp.dot(xb, w[_H:, :], preferred_element_type=F32))

    return pl.pallas_call(
        body,
        grid=(N // BN,),
        in_specs=[pl.BlockSpec((2, BN, _D), lambda i: (0, i, 0)),
                  pl.BlockSpec((BN, D), lambda i: (i, 0)),
                  pl.BlockSpec((_H + D, D), lambda i: (0, 0))],
        out_specs=pl.BlockSpec((BN, D), lambda i: (i, 0)),
        out_shape=jax.ShapeDtypeStruct((N, D), F32),
    )(parts, x, We1)


def _tc_reduce(o16, be2):
    E = o16.shape[0]
    BE = 2560
    G = E // BE

    def body(o_ref, b_ref, r_ref):
        z = jnp.sum(o_ref[...], axis=1, keepdims=True) + b_ref[...]
        r_ref[...] = 1.0 / (1.0 + jnp.exp(-z))

    return pl.pallas_call(
        body,
        grid=(G,),
        in_specs=[pl.BlockSpec((BE, _L), lambda i: (i, 0)),
                  pl.BlockSpec((1, 1), lambda i: (0, 0))],
        out_specs=pl.BlockSpec((BE, 1), lambda i: (i, 0)),
        out_shape=jax.ShapeDtypeStruct((E, 1), F32),
    )(o16, be2)


def _affine(ssum, ssq, g, bt, count):
    m = ssum / count
    v = ssq / count - m * m
    s = g * lax.rsqrt(v + 1e-5)
    t = bt - m * s
    return jnp.stack([s, t])


def kernel(x, edge_index, W1, b1, g1, bt1, W2, b2, g2, bt2,
           W3, b3, g3, bt3, We1, be1, We2, be2):
    N, D = x.shape
    E = edge_index.shape[1]
    EPW = E // _NW
    src = edge_index[0]
    dst = edge_index[1]

    AB = _tc_pre(x, W1)
    h1, st1f = _sc_gather(E, N)(AB, src, dst)
    st1p = st1f.reshape(_NW, _D)
    cE = jnp.float32(E)
    st1 = _affine(jnp.sum(st1p[:, :_H], axis=0),
                  jnp.sum(st1p[:, _H:], axis=0), g1, bt1, cE)
    h2, ss2, sq2 = _tc_mid(h1, st1, W2)
    st2 = _affine(ss2, sq2, g2, bt2, cE)
    h3, ss3, sq3 = _tc_mid(h2, st2, W3, pad_count=True)
    st3 = _affine(ss3, sq3, g3, bt3, cE)

    dst3 = dst.reshape(_NW, EPW // _K, _K)
    parts = _sc_scatter(E, N)(h3, dst3, jnp.concatenate([st3[0], st3[1]]))
    C = _tc_combine(parts, x, We1)
    prm = jnp.concatenate([be1, We2.reshape(-1), be2.reshape(-1),
                           jnp.zeros((15,), F32)])
    o16 = _sc_final(E, N)(C, src, dst, prm)
    return _tc_reduce(o16, be2.reshape(1, 1))


# speedup vs baseline: 1.2739x; 1.2739x over previous
import functools

import jax
import jax.numpy as jnp
from jax import lax
from jax.experimental import pallas as pl
from jax.experimental.pallas import tpu as pltpu
from jax.experimental.pallas import tpu_sc as plsc

F32 = jnp.float32
_NC = 2
_NS = 16
_NW = _NC * _NS
_L = 16
_H = 64
_D = 128
_K = 80


def _mesh():
    return plsc.VectorSubcoreMesh(core_axis_name="c", subcore_axis_name="s")


def _sc_gather(E, N):
    EPW = E // _NW
    NBLK = EPW // _K
    PAIRS = (NBLK - 1) // 2

    @functools.partial(
        pl.kernel,
        out_type=[
            jax.ShapeDtypeStruct((E, _H), F32),
            jax.ShapeDtypeStruct((_NW * _D,), F32),
        ],
        mesh=_mesh(),
        scratch_types=[
            pltpu.VMEM((EPW,), jnp.int32),
            pltpu.VMEM((EPW,), jnp.int32),
            pltpu.VMEM((_K, _D), F32),
            pltpu.VMEM((_K, _D), F32),
            pltpu.VMEM((_K, _D), F32),
            pltpu.VMEM((_K, _D), F32),
            pltpu.VMEM((_K, _H), F32),
            pltpu.VMEM((_K, _H), F32),
            pltpu.VMEM((_D,), F32),
            pltpu.SemaphoreType.DMA,
            pltpu.SemaphoreType.DMA,
            pltpu.SemaphoreType.DMA,
            pltpu.SemaphoreType.DMA,
        ],
    )
    def k(ab, srcv, dstv, h_out, st_out,
          isv, idv, ra0, ra1, rb0, rb1, hb0, hb1, sb, sA0, sA1, sO0, sO1):
        cid = lax.axis_index("c")
        sid = lax.axis_index("s")
        wid = sid * _NC + cid
        base = wid * EPW
        zero = jnp.zeros((_L,), F32)

        pltpu.sync_copy(srcv.at[pl.ds(base, EPW)], isv)
        pltpu.sync_copy(dstv.at[pl.ds(base, EPW)], idv)

        def issue(b, ra, rb, sem):
            pltpu.async_copy(ab.at[idv.at[pl.ds(b * _K, _K)]], ra, sem)
            pltpu.async_copy(ab.at[isv.at[pl.ds(b * _K, _K)]], rb, sem)

        def drain_g(ra, rb, sem):
            pltpu.make_async_copy(ab.at[pl.ds(0, _K)], ra, sem).wait()
            pltpu.make_async_copy(ab.at[pl.ds(0, _K)], rb, sem).wait()

        def out(b, hb, sem):
            pltpu.async_copy(hb, h_out.at[pl.ds(base + b * _K, _K)], sem)

        def drain_o(hb, sem):
            pltpu.make_async_copy(hb, h_out.at[pl.ds(base, _K)], sem).wait()

        def compute(ra, rb, hb, carry):
            def row(r, c8):
                acc = list(c8)
                for c in range(4):
                    h = (ra[r, pl.ds(c * _L, _L)]
                         + rb[r, pl.ds(_H + c * _L, _L)])
                    hb[r, pl.ds(c * _L, _L)] = h
                    acc[c] = acc[c] + h
                    acc[4 + c] = acc[4 + c] + h * h
                return tuple(acc)

            return lax.fori_loop(0, _K, row, carry, unroll=8)

        issue(0, ra0, rb0, sA0)
        out(0, hb0, sO0)
        out(1, hb1, sO1)

        def pair(t, carry):
            a = 2 * t
            issue(a + 1, ra1, rb1, sA1)
            drain_g(ra0, rb0, sA0)
            drain_o(hb0, sO0)
            carry = compute(ra0, rb0, hb0, carry)
            out(a, hb0, sO0)
            issue(a + 2, ra0, rb0, sA0)
            drain_g(ra1, rb1, sA1)
            drain_o(hb1, sO1)
            carry = compute(ra1, rb1, hb1, carry)
            out(a + 1, hb1, sO1)
            return carry

        carry = lax.fori_loop(0, PAIRS, pair, (zero,) * 8)
        drain_g(ra0, rb0, sA0)
        drain_o(hb0, sO0)
        carry = compute(ra0, rb0, hb0, carry)
        out(NBLK - 1, hb0, sO0)
        drain_o(hb0, sO0)
        drain_o(hb1, sO1)

        for c in range(4):
            sb[pl.ds(c * _L, _L)] = carry[c]
            sb[pl.ds(_H + c * _L, _L)] = carry[4 + c]
        pltpu.sync_copy(sb, st_out.at[pl.ds(wid * _D, _D)])

    return k


def _sc_scatter(E, N):
    EPW = E // _NW
    NBLK = EPW // _K
    PAIRS = (NBLK - 1) // 2
    CH = 40
    NCHUNK = N // CH
    TPW = (NCHUNK + _NS - 1) // _NS

    @functools.partial(
        pl.kernel,
        out_type=jax.ShapeDtypeStruct((_NC, N, _D), F32),
        mesh=_mesh(),
        scratch_types=[
            pltpu.VMEM_SHARED((N, _D), F32),
            pltpu.VMEM((NBLK, _K), jnp.int32),
            pltpu.VMEM((_K, _D), F32),
            pltpu.VMEM((_K, _D), F32),
            pltpu.VMEM((CH, _D), F32),
            pltpu.VMEM((_D,), F32),
            pltpu.SemaphoreType.DMA,
            pltpu.SemaphoreType.DMA,
        ],
    )
    def k(h3, dst3, st_hbm, part,
          acc, idx3, pb0, pb1, zbuf, stv, sS0, sS1):
        cid = lax.axis_index("c")
        sid = lax.axis_index("s")
        wid = sid * _NC + cid
        base = wid * EPW
        pltpu.sync_copy(st_hbm, stv)
        pltpu.sync_copy(dst3.at[wid], idx3)

        zv = jnp.zeros((_L,), F32)

        def zrow(r, _):
            for c in range(_D // _L):
                zbuf[r, pl.ds(c * _L, _L)] = zv
            return 0

        lax.fori_loop(0, CH, zrow, 0)
        for t in range(TPW):
            c = sid * TPW + t

            @pl.when(c < NCHUNK)
            def _():
                pltpu.sync_copy(zbuf, acc.at[pl.ds(c * CH, CH)])

        def pzrow(r, _):
            for c in range(_D // _L):
                pb0[r, pl.ds(c * _L, _L)] = zv
                pb1[r, pl.ds(c * _L, _L)] = zv
            return 0

        lax.fori_loop(0, _K, pzrow, 0)
        plsc.subcore_barrier()

        svec = [stv[pl.ds(c * _L, _L)] for c in range(4)]
        tvec = [stv[pl.ds(_H + c * _L, _L)] for c in range(4)]

        def scat(b, pb, sem):
            pltpu.async_copy(pb, acc.at[idx3.at[b]], sem, add=True)

        def drain_s(pb, sem):
            pltpu.make_async_copy(pb, acc.at[idx3.at[0]], sem).wait()

        def block(b, pb, sem):
            drain_s(pb, sem)
            pltpu.sync_copy(h3.at[pl.ds(base + b * _K, _K)], pb)

            def row(r, _2):
                for c in range(4):
                    v = pb[r, pl.ds(c * _L, _L)]
                    pb[r, pl.ds(c * _L, _L)] = jnp.maximum(
                        v * svec[c] + tvec[c], 0.0)
                return 0

            lax.fori_loop(0, _K, row, 0, unroll=8)
            scat(b, pb, sem)

        scat(0, pb0, sS0)
        scat(1, pb1, sS1)

        def pair(t, _):
            a = 2 * t
            block(a, pb0, sS0)
            block(a + 1, pb1, sS1)
            return 0

        lax.fori_loop(0, PAIRS, pair, 0)
        block(NBLK - 1, pb0, sS0)
        drain_s(pb0, sS0)
        drain_s(pb1, sS1)
        plsc.subcore_barrier()

        for t in range(TPW):
            c = sid * TPW + t

            @pl.when(c < NCHUNK)
            def _():
                pltpu.sync_copy(acc.at[pl.ds(c * CH, CH)], zbuf)
                pltpu.sync_copy(zbuf, part.at[cid, pl.ds(c * CH, CH)])

    return k


def _sc_final(E, N):
    EPW = E // _NW
    NBLK = EPW // _K
    PAIRS = (NBLK - 1) // 2

    @functools.partial(
        pl.kernel,
        out_type=jax.ShapeDtypeStruct((E,), F32),
        mesh=_mesh(),
        scratch_types=[
            pltpu.VMEM((EPW,), jnp.int32),
            pltpu.VMEM((EPW,), jnp.int32),
            pltpu.VMEM((_K, _D), F32),
            pltpu.VMEM((_K, _D), F32),
            pltpu.VMEM((_K, _D), F32),
            pltpu.VMEM((_K, _D), F32),
            pltpu.VMEM((_K + _L,), F32),
            pltpu.VMEM((_K + _L,), F32),
            pltpu.VMEM((272,), F32),
            pltpu.SemaphoreType.DMA,
            pltpu.SemaphoreType.DMA,
            pltpu.SemaphoreType.DMA,
            pltpu.SemaphoreType.DMA,
        ],
    )
    def k(c_hbm, srcv, dstv, prm, o_hbm,
          isv, idv, rs0, rs1, rd0, rd1, ob0, ob1, pv, sA0, sA1, sO0, sO1):
        cid = lax.axis_index("c")
        sid = lax.axis_index("s")
        wid = sid * _NC + cid
        base = wid * EPW

        pltpu.sync_copy(prm, pv)
        pltpu.sync_copy(srcv.at[pl.ds(base, EPW)], isv)
        pltpu.sync_copy(dstv.at[pl.ds(base, EPW)], idv)
        lanes = lax.iota(jnp.int32, _L)
        bev = [pv[pl.ds(c * _L, _L)] for c in range(_D // _L)]
        wev = [pv[pl.ds(_D + c * _L, _L)] for c in range(_D // _L)]
        be2s = pv[pl.ds(2 * _D, _L)][0]

        def issue(b, rs, rd, sem):
            pltpu.async_copy(c_hbm.at[isv.at[pl.ds(b * _K, _K)]], rs, sem)
            pltpu.async_copy(c_hbm.at[idv.at[pl.ds(b * _K, _K)]], rd, sem)

        def drain_g(rs, rd, sem):
            pltpu.make_async_copy(c_hbm.at[pl.ds(0, _K)], rs, sem).wait()
            pltpu.make_async_copy(c_hbm.at[pl.ds(0, _K)], rd, sem).wait()

        def out(b, ob, sem):
            pltpu.async_copy(ob.at[pl.ds(0, _K)],
                             o_hbm.at[pl.ds(base + b * _K, _K)], sem)

        def drain_o(ob, sem):
            pltpu.make_async_copy(ob.at[pl.ds(0, _K)],
                                  o_hbm.at[pl.ds(base, _K)], sem).wait()

        def compute(rs, rd, ob):
            def row(r, ovec):
                accs = []
                for c in range(_D // _L):
                    dv = rs[r, pl.ds(c * _L, _L)] - rd[r, pl.ds(c * _L, _L)]
                    dv = jnp.maximum(dv + bev[c], 0.0)
                    accs.append(dv * wev[c])
                while len(accs) > 1:
                    accs = [a + b for a, b in zip(accs[::2], accs[1::2])]
                parts = [accs[0][l] for l in range(_L)]
                while len(parts) > 1:
                    parts = [p + q for p, q in zip(parts[::2], parts[1::2])]
                z = parts[0] + be2s
                lr = lax.rem(r, _L)
                ovec = jnp.where(lanes == lr, z, ovec)

                @pl.when((lr == _L - 1) | (r == _K - 1))
                def _():
                    ob[pl.ds(r - lr, _L)] = 1.0 / (1.0 + jnp.exp(-ovec))

                return ovec

            lax.fori_loop(0, _K, row, jnp.zeros((_L,), F32), unroll=8)

        issue(0, rs0, rd0, sA0)
        out(0, ob0, sO0)
        out(1, ob1, sO1)

        def pair(t, _):
            a = 2 * t
            issue(a + 1, rs1, rd1, sA1)
            drain_g(rs0, rd0, sA0)
            drain_o(ob0, sO0)
            compute(rs0, rd0, ob0)
            out(a, ob0, sO0)
            issue(a + 2, rs0, rd0, sA0)
            drain_g(rs1, rd1, sA1)
            drain_o(ob1, sO1)
            compute(rs1, rd1, ob1)
            out(a + 1, ob1, sO1)
            return 0

        lax.fori_loop(0, PAIRS, pair, 0)
        drain_g(rs0, rd0, sA0)
        drain_o(ob0, sO0)
        compute(rs0, rd0, ob0)
        out(NBLK - 1, ob0, sO0)
        drain_o(ob0, sO0)
        drain_o(ob1, sO1)

    return k


def _tc_pre(x, W1):
    N, D = x.shape
    BN = 2000

    def body(x_ref, w_ref, ab_ref):
        w = w_ref[...]
        wa = w[:D, :]
        wb = w[D:, :]
        xb = x_ref[...]
        ab_ref[...] = jnp.concatenate(
            [jnp.dot(xb, wa - wb, preferred_element_type=F32),
             jnp.dot(xb, wb, preferred_element_type=F32)], axis=1)

    return pl.pallas_call(
        body,
        grid=(N // BN,),
        in_specs=[pl.BlockSpec((BN, D), lambda i: (i, 0)),
                  pl.BlockSpec((2 * D, _H), lambda i: (0, 0))],
        out_specs=pl.BlockSpec((BN, 2 * _H), lambda i: (i, 0)),
        out_shape=jax.ShapeDtypeStruct((N, 2 * _H), F32),
    )(x, W1)


def _tc_mid(h, st, W, pad_count=False):
    E = h.shape[0]
    BE = 2560
    G = E // BE
    OW = _D if pad_count else _H

    def body(h_ref, st_ref, w_ref, o_ref, ss_ref, sq_ref):
        i = pl.program_id(0)
        stb = st_ref[...]
        hb = h_ref[...].astype(F32)
        h1 = jnp.maximum(hb * stb[0:1, :] + stb[1:2, :], 0.0)
        o = jnp.dot(h1, w_ref[...], preferred_element_type=F32)
        if pad_count:
            cols = lax.broadcasted_iota(jnp.int32, (BE, _H), 1)
            cc = jnp.where(cols == 0, jnp.float32(1), jnp.float32(0))
            o_ref[...] = jnp.concatenate([o, cc], axis=1)
        else:
            o_ref[...] = o.astype(jnp.bfloat16)
        s = jnp.broadcast_to(jnp.sum(o, axis=0, keepdims=True), (8, _H))
        q = jnp.broadcast_to(jnp.sum(o * o, axis=0, keepdims=True), (8, _H))

        @pl.when(i == 0)
        def _():
            ss_ref[...] = s
            sq_ref[...] = q

        @pl.when(i > 0)
        def _():
            ss_ref[...] = ss_ref[...] + s
            sq_ref[...] = sq_ref[...] + q

    out, ss, sq = pl.pallas_call(
        body,
        grid=(G,),
        in_specs=[pl.BlockSpec((BE, _H), lambda i: (i, 0)),
                  pl.BlockSpec((2, _H), lambda i: (0, 0)),
                  pl.BlockSpec((_H, _H), lambda i: (0, 0))],
        out_specs=[pl.BlockSpec((BE, OW), lambda i: (i, 0)),
                   pl.BlockSpec((8, _H), lambda i: (0, 0)),
                   pl.BlockSpec((8, _H), lambda i: (0, 0))],
        out_shape=[jax.ShapeDtypeStruct((E, OW),
                                        F32 if pad_count else jnp.bfloat16),
                   jax.ShapeDtypeStruct((8, _H), F32),
                   jax.ShapeDtypeStruct((8, _H), F32)],
    )(h, st, W)
    return out, ss[0], sq[0]


def _tc_combine(parts, x, We1):
    N, D = x.shape
    BN = 2000

    def body(p_ref, x_ref, w_ref, c_ref):
        p = p_ref[...]
        ps = p[0] + p[1]
        cnt = jnp.maximum(ps[:, _H:_H + 1], 1.0)
        agg = jnp.maximum(ps[:, :_H] / cnt, 0.0)
        xb = jnp.maximum(x_ref[...], 0.0)
        w = w_ref[...]
        c_ref[...] = (jnp.dot(agg, w[:_H, :], preferred_element_type=F32)
                      + jnp.dot(xb, w[_H:, :], preferred_element_type=F32))

    return pl.pallas_call(
        body,
        grid=(N // BN,),
        in_specs=[pl.BlockSpec((2, BN, _D), lambda i: (0, i, 0)),
                  pl.BlockSpec((BN, D), lambda i: (i, 0)),
                  pl.BlockSpec((_H + D, D), lambda i: (0, 0))],
        out_specs=pl.BlockSpec((BN, D), lambda i: (i, 0)),
        out_shape=jax.ShapeDtypeStruct((N, D), F32),
    )(parts, x, We1)


def _affine(ssum, ssq, g, bt, count):
    m = ssum / count
    v = ssq / count - m * m
    s = g * lax.rsqrt(v + 1e-5)
    t = bt - m * s
    return jnp.stack([s, t])


def kernel(x, edge_index, W1, b1, g1, bt1, W2, b2, g2, bt2,
           W3, b3, g3, bt3, We1, be1, We2, be2):
    N, D = x.shape
    E = edge_index.shape[1]
    EPW = E // _NW
    src = edge_index[0]
    dst = edge_index[1]

    AB = _tc_pre(x, W1)
    h1, st1f = _sc_gather(E, N)(AB, src, dst)
    st1p = st1f.reshape(_NW, _D)
    cE = jnp.float32(E)
    st1 = _affine(jnp.sum(st1p[:, :_H], axis=0),
                  jnp.sum(st1p[:, _H:], axis=0), g1, bt1, cE)
    h2, ss2, sq2 = _tc_mid(h1, st1, W2)
    st2 = _affine(ss2, sq2, g2, bt2, cE)
    h3, ss3, sq3 = _tc_mid(h2, st2, W3, pad_count=True)
    st3 = _affine(ss3, sq3, g3, bt3, cE)

    dst3 = dst.reshape(_NW, EPW // _K, _K)
    parts = _sc_scatter(E, N)(h3, dst3, jnp.concatenate([st3[0], st3[1]]))
    C = _tc_combine(parts, x, We1)
    prm = jnp.concatenate([be1, We2.reshape(-1), be2.reshape(-1),
                           jnp.zeros((15,), F32)])
    o = _sc_final(E, N)(C, src, dst, prm)
    return o.reshape(E, 1)


# speedup vs baseline: 1.4189x; 1.1138x over previous
import functools

import jax
import jax.numpy as jnp
from jax import lax
from jax.experimental import pallas as pl
from jax.experimental.pallas import tpu as pltpu
from jax.experimental.pallas import tpu_sc as plsc

F32 = jnp.float32
_NC = 2
_NS = 16
_NW = _NC * _NS
_L = 16
_H = 64
_D = 128
_K = 80


def _mesh():
    return plsc.VectorSubcoreMesh(core_axis_name="c", subcore_axis_name="s")


def _sc_gather(E, N):
    EPW = E // _NW
    NBLK = EPW // _K
    PAIRS = (NBLK - 1) // 2

    @functools.partial(
        pl.kernel,
        out_type=[
            jax.ShapeDtypeStruct((E, _H), F32),
            jax.ShapeDtypeStruct((_NW * _D,), F32),
        ],
        mesh=_mesh(),
        scratch_types=[
            pltpu.VMEM((EPW,), jnp.int32),
            pltpu.VMEM((EPW,), jnp.int32),
            pltpu.VMEM((_K, _D), F32),
            pltpu.VMEM((_K, _D), F32),
            pltpu.VMEM((_K, _D), F32),
            pltpu.VMEM((_K, _D), F32),
            pltpu.VMEM((_K, _H), F32),
            pltpu.VMEM((_K, _H), F32),
            pltpu.VMEM((_D,), F32),
            pltpu.SemaphoreType.DMA,
            pltpu.SemaphoreType.DMA,
            pltpu.SemaphoreType.DMA,
            pltpu.SemaphoreType.DMA,
        ],
    )
    def k(ab, srcv, dstv, h_out, st_out,
          isv, idv, ra0, ra1, rb0, rb1, hb0, hb1, sb, sA0, sA1, sO0, sO1):
        cid = lax.axis_index("c")
        sid = lax.axis_index("s")
        wid = sid * _NC + cid
        base = wid * EPW
        zero = jnp.zeros((_L,), F32)

        pltpu.sync_copy(srcv.at[pl.ds(base, EPW)], isv)
        pltpu.sync_copy(dstv.at[pl.ds(base, EPW)], idv)

        def issue(b, ra, rb, sem):
            pltpu.async_copy(ab.at[idv.at[pl.ds(b * _K, _K)]], ra, sem)
            pltpu.async_copy(ab.at[isv.at[pl.ds(b * _K, _K)]], rb, sem)

        def drain_g(ra, rb, sem):
            pltpu.make_async_copy(ab.at[pl.ds(0, _K)], ra, sem).wait()
            pltpu.make_async_copy(ab.at[pl.ds(0, _K)], rb, sem).wait()

        def out(b, hb, sem):
            pltpu.async_copy(hb, h_out.at[pl.ds(base + b * _K, _K)], sem)

        def drain_o(hb, sem):
            pltpu.make_async_copy(hb, h_out.at[pl.ds(base, _K)], sem).wait()

        def compute(ra, rb, hb, carry):
            def row(r, c8):
                acc = list(c8)
                for c in range(4):
                    h = (ra[r, pl.ds(c * _L, _L)]
                         + rb[r, pl.ds(_H + c * _L, _L)])
                    hb[r, pl.ds(c * _L, _L)] = h
                    acc[c] = acc[c] + h
                    acc[4 + c] = acc[4 + c] + h * h
                return tuple(acc)

            return lax.fori_loop(0, _K, row, carry, unroll=8)

        issue(0, ra0, rb0, sA0)
        out(0, hb0, sO0)
        out(1, hb1, sO1)

        def pair(t, carry):
            a = 2 * t
            issue(a + 1, ra1, rb1, sA1)
            drain_g(ra0, rb0, sA0)
            drain_o(hb0, sO0)
            carry = compute(ra0, rb0, hb0, carry)
            out(a, hb0, sO0)
            issue(a + 2, ra0, rb0, sA0)
            drain_g(ra1, rb1, sA1)
            drain_o(hb1, sO1)
            carry = compute(ra1, rb1, hb1, carry)
            out(a + 1, hb1, sO1)
            return carry

        carry = lax.fori_loop(0, PAIRS, pair, (zero,) * 8)
        drain_g(ra0, rb0, sA0)
        drain_o(hb0, sO0)
        carry = compute(ra0, rb0, hb0, carry)
        out(NBLK - 1, hb0, sO0)
        drain_o(hb0, sO0)
        drain_o(hb1, sO1)

        for c in range(4):
            sb[pl.ds(c * _L, _L)] = carry[c]
            sb[pl.ds(_H + c * _L, _L)] = carry[4 + c]
        pltpu.sync_copy(sb, st_out.at[pl.ds(wid * _D, _D)])

    return k


def _sc_scatter(E, N):
    EPW = E // _NW
    NBLK = EPW // _K
    PAIRS = (NBLK - 1) // 2
    CH = 40
    NCHUNK = N // CH
    TPW = (NCHUNK + _NS - 1) // _NS

    @functools.partial(
        pl.kernel,
        out_type=jax.ShapeDtypeStruct((_NC, N, _D), F32),
        mesh=_mesh(),
        scratch_types=[
            pltpu.VMEM_SHARED((N, _D), F32),
            pltpu.VMEM((NBLK, _K), jnp.int32),
            pltpu.VMEM((_K, _D), F32),
            pltpu.VMEM((_K, _D), F32),
            pltpu.VMEM((CH, _D), F32),
            pltpu.VMEM((_D,), F32),
            pltpu.SemaphoreType.DMA,
            pltpu.SemaphoreType.DMA,
        ],
    )
    def k(h3, dst3, st_hbm, part,
          acc, idx3, pb0, pb1, zbuf, stv, sS0, sS1):
        cid = lax.axis_index("c")
        sid = lax.axis_index("s")
        wid = sid * _NC + cid
        base = wid * EPW
        pltpu.sync_copy(st_hbm, stv)
        pltpu.sync_copy(dst3.at[wid], idx3)

        zv = jnp.zeros((_L,), F32)

        def zrow(r, _):
            for c in range(_D // _L):
                zbuf[r, pl.ds(c * _L, _L)] = zv
            return 0

        lax.fori_loop(0, CH, zrow, 0)
        for t in range(TPW):
            c = sid * TPW + t

            @pl.when(c < NCHUNK)
            def _():
                pltpu.sync_copy(zbuf, acc.at[pl.ds(c * CH, CH)])

        def pzrow(r, _):
            for c in range(_D // _L):
                pb0[r, pl.ds(c * _L, _L)] = zv
                pb1[r, pl.ds(c * _L, _L)] = zv
            return 0

        lax.fori_loop(0, _K, pzrow, 0)
        plsc.subcore_barrier()

        svec = [stv[pl.ds(c * _L, _L)] for c in range(4)]
        tvec = [stv[pl.ds(_H + c * _L, _L)] for c in range(4)]

        def scat(b, pb, sem):
            pltpu.async_copy(pb, acc.at[idx3.at[b]], sem, add=True)

        def drain_s(pb, sem):
            pltpu.make_async_copy(pb, acc.at[idx3.at[0]], sem).wait()

        def block(b, pb, sem):
            drain_s(pb, sem)
            pltpu.sync_copy(h3.at[pl.ds(base + b * _K, _K)], pb)

            def row(r, _2):
                for c in range(4):
                    v = pb[r, pl.ds(c * _L, _L)]
                    pb[r, pl.ds(c * _L, _L)] = jnp.maximum(
                        v * svec[c] + tvec[c], 0.0)
                return 0

            lax.fori_loop(0, _K, row, 0, unroll=8)
            scat(b, pb, sem)

        scat(0, pb0, sS0)
        scat(1, pb1, sS1)

        def pair(t, _):
            a = 2 * t
            block(a, pb0, sS0)
            block(a + 1, pb1, sS1)
            return 0

        lax.fori_loop(0, PAIRS, pair, 0)
        block(NBLK - 1, pb0, sS0)
        drain_s(pb0, sS0)
        drain_s(pb1, sS1)
        plsc.subcore_barrier()

        for t in range(TPW):
            c = sid * TPW + t

            @pl.when(c < NCHUNK)
            def _():
                pltpu.sync_copy(acc.at[pl.ds(c * CH, CH)], zbuf)
                pltpu.sync_copy(zbuf, part.at[cid, pl.ds(c * CH, CH)])

    return k


def _sc_final(E, N):
    EPW = E // _NW
    NBLK = EPW // _K
    PAIRS = (NBLK - 1) // 2

    @functools.partial(
        pl.kernel,
        out_type=jax.ShapeDtypeStruct((E,), F32),
        mesh=_mesh(),
        scratch_types=[
            pltpu.VMEM((EPW,), jnp.int32),
            pltpu.VMEM((EPW,), jnp.int32),
            pltpu.VMEM((_K, _D), F32),
            pltpu.VMEM((_K, _D), F32),
            pltpu.VMEM((_K, _D), F32),
            pltpu.VMEM((_K, _D), F32),
            pltpu.VMEM((_K + _L,), F32),
            pltpu.VMEM((_K + _L,), F32),
            pltpu.VMEM((272,), F32),
            pltpu.SemaphoreType.DMA,
            pltpu.SemaphoreType.DMA,
            pltpu.SemaphoreType.DMA,
            pltpu.SemaphoreType.DMA,
        ],
    )
    def k(c_hbm, srcv, dstv, prm, o_hbm,
          isv, idv, rs0, rs1, rd0, rd1, ob0, ob1, pv, sA0, sA1, sO0, sO1):
        cid = lax.axis_index("c")
        sid = lax.axis_index("s")
        wid = sid * _NC + cid
        base = wid * EPW

        pltpu.sync_copy(prm, pv)
        pltpu.sync_copy(srcv.at[pl.ds(base, EPW)], isv)
        pltpu.sync_copy(dstv.at[pl.ds(base, EPW)], idv)
        lanes = lax.iota(jnp.int32, _L)
        bev = [pv[pl.ds(c * _L, _L)] for c in range(_D // _L)]
        wev = [pv[pl.ds(_D + c * _L, _L)] for c in range(_D // _L)]
        be2s = pv[pl.ds(2 * _D, _L)][0]

        def issue(b, rs, rd, sem):
            pltpu.async_copy(c_hbm.at[isv.at[pl.ds(b * _K, _K)]], rs, sem)
            pltpu.async_copy(c_hbm.at[idv.at[pl.ds(b * _K, _K)]], rd, sem)

        def drain_g(rs, rd, sem):
            pltpu.make_async_copy(c_hbm.at[pl.ds(0, _K)], rs, sem).wait()
            pltpu.make_async_copy(c_hbm.at[pl.ds(0, _K)], rd, sem).wait()

        def out(b, ob, sem):
            pltpu.async_copy(ob.at[pl.ds(0, _K)],
                             o_hbm.at[pl.ds(base + b * _K, _K)], sem)

        def drain_o(ob, sem):
            pltpu.make_async_copy(ob.at[pl.ds(0, _K)],
                                  o_hbm.at[pl.ds(base, _K)], sem).wait()

        def compute(rs, rd, ob):
            def row(r, ovec):
                accs = []
                for c in range(_D // _L):
                    dv = rs[r, pl.ds(c * _L, _L)] - rd[r, pl.ds(c * _L, _L)]
                    dv = jnp.maximum(dv + bev[c], 0.0)
                    accs.append(dv * wev[c])
                while len(accs) > 1:
                    accs = [a + b for a, b in zip(accs[::2], accs[1::2])]
                parts = [accs[0][l] for l in range(_L)]
                while len(parts) > 1:
                    parts = [p + q for p, q in zip(parts[::2], parts[1::2])]
                z = parts[0] + be2s
                lr = lax.rem(r, _L)
                ovec = jnp.where(lanes == lr, z, ovec)

                @pl.when((lr == _L - 1) | (r == _K - 1))
                def _():
                    ob[pl.ds(r - lr, _L)] = 1.0 / (1.0 + jnp.exp(-ovec))

                return ovec

            lax.fori_loop(0, _K, row, jnp.zeros((_L,), F32), unroll=8)

        issue(0, rs0, rd0, sA0)
        out(0, ob0, sO0)
        out(1, ob1, sO1)

        def pair(t, _):
            a = 2 * t
            issue(a + 1, rs1, rd1, sA1)
            drain_g(rs0, rd0, sA0)
            drain_o(ob0, sO0)
            compute(rs0, rd0, ob0)
            out(a, ob0, sO0)
            issue(a + 2, rs0, rd0, sA0)
            drain_g(rs1, rd1, sA1)
            drain_o(ob1, sO1)
            compute(rs1, rd1, ob1)
            out(a + 1, ob1, sO1)
            return 0

        lax.fori_loop(0, PAIRS, pair, 0)
        drain_g(rs0, rd0, sA0)
        drain_o(ob0, sO0)
        compute(rs0, rd0, ob0)
        out(NBLK - 1, ob0, sO0)
        drain_o(ob0, sO0)
        drain_o(ob1, sO1)

    return k


def _tc_pre(x, W1):
    N, D = x.shape
    BN = 2000

    def body(x_ref, w_ref, ab_ref):
        w = w_ref[...]
        wa = w[:D, :]
        wb = w[D:, :]
        xb = x_ref[...]
        ab_ref[...] = jnp.concatenate(
            [jnp.dot(xb, wa - wb, preferred_element_type=F32),
             jnp.dot(xb, wb, preferred_element_type=F32)], axis=1)

    return pl.pallas_call(
        body,
        grid=(N // BN,),
        in_specs=[pl.BlockSpec((BN, D), lambda i: (i, 0)),
                  pl.BlockSpec((2 * D, _H), lambda i: (0, 0))],
        out_specs=pl.BlockSpec((BN, 2 * _H), lambda i: (i, 0)),
        out_shape=jax.ShapeDtypeStruct((N, 2 * _H), F32),
    )(x, W1)


def _tc_mid(h, st, W, pad_count=False):
    E = h.shape[0]
    BE = 8000
    G = E // BE
    OW = _D if pad_count else _H

    def body(h_ref, st_ref, w_ref, o_ref, ss_ref, sq_ref):
        i = pl.program_id(0)
        stb = st_ref[...]
        hb = h_ref[...].astype(F32)
        h1 = jnp.maximum(hb * stb[0:1, :] + stb[1:2, :], 0.0)
        o = jnp.dot(h1, w_ref[...], preferred_element_type=F32)
        if pad_count:
            cols = lax.broadcasted_iota(jnp.int32, (BE, _H), 1)
            cc = jnp.where(cols == 0, jnp.float32(1), jnp.float32(0))
            o_ref[...] = jnp.concatenate([o, cc], axis=1)
        else:
            o_ref[...] = o.astype(jnp.bfloat16)
        s = jnp.broadcast_to(jnp.sum(o, axis=0, keepdims=True), (8, _H))
        q = jnp.broadcast_to(jnp.sum(o * o, axis=0, keepdims=True), (8, _H))

        @pl.when(i == 0)
        def _():
            ss_ref[...] = s
            sq_ref[...] = q

        @pl.when(i > 0)
        def _():
            ss_ref[...] = ss_ref[...] + s
            sq_ref[...] = sq_ref[...] + q

    out, ss, sq = pl.pallas_call(
        body,
        grid=(G,),
        in_specs=[pl.BlockSpec((BE, _H), lambda i: (i, 0)),
                  pl.BlockSpec((2, _H), lambda i: (0, 0)),
                  pl.BlockSpec((_H, _H), lambda i: (0, 0))],
        out_specs=[pl.BlockSpec((BE, OW), lambda i: (i, 0)),
                   pl.BlockSpec((8, _H), lambda i: (0, 0)),
                   pl.BlockSpec((8, _H), lambda i: (0, 0))],
        out_shape=[jax.ShapeDtypeStruct((E, OW),
                                        F32 if pad_count else jnp.bfloat16),
                   jax.ShapeDtypeStruct((8, _H), F32),
                   jax.ShapeDtypeStruct((8, _H), F32)],
    )(h, st, W)
    return out, ss[0], sq[0]


def _tc_combine(parts, x, We1):
    N, D = x.shape
    BN = 2000

    def body(p_ref, x_ref, w_ref, c_ref):
        p = p_ref[...]
        ps = p[0] + p[1]
        cnt = jnp.maximum(ps[:, _H:_H + 1], 1.0)
        agg = jnp.maximum(ps[:, :_H] / cnt, 0.0)
        xb = jnp.maximum(x_ref[...], 0.0)
        w = w_ref[...]
        c_ref[...] = (jnp.dot(agg, w[:_H, :], preferred_element_type=F32)
                      + jnp.dot(xb, w[_H:, :], preferred_element_type=F32))

    return pl.pallas_call(
        body,
        grid=(N // BN,),
        in_specs=[pl.BlockSpec((2, BN, _D), lambda i: (0, i, 0)),
                  pl.BlockSpec((BN, D), lambda i: (i, 0)),
                  pl.BlockSpec((_H + D, D), lambda i: (0, 0))],
        out_specs=pl.BlockSpec((BN, D), lambda i: (i, 0)),
        out_shape=jax.ShapeDtypeStruct((N, D), F32),
    )(parts, x, We1)


def _affine(ssum, ssq, g, bt, count):
    m = ssum / count
    v = ssq / count - m * m
    s = g * lax.rsqrt(v + 1e-5)
    t = bt - m * s
    return jnp.stack([s, t])


def kernel(x, edge_index, W1, b1, g1, bt1, W2, b2, g2, bt2,
           W3, b3, g3, bt3, We1, be1, We2, be2):
    N, D = x.shape
    E = edge_index.shape[1]
    EPW = E // _NW
    src = edge_index[0]
    dst = edge_index[1]

    AB = _tc_pre(x, W1)
    h1, st1f = _sc_gather(E, N)(AB, src, dst)
    st1p = st1f.reshape(_NW, _D)
    cE = jnp.float32(E)
    st1 = _affine(jnp.sum(st1p[:, :_H], axis=0),
                  jnp.sum(st1p[:, _H:], axis=0), g1, bt1, cE)
    h2, ss2, sq2 = _tc_mid(h1, st1, W2)
    st2 = _affine(ss2, sq2, g2, bt2, cE)
    h3, ss3, sq3 = _tc_mid(h2, st2, W3, pad_count=True)
    st3 = _affine(ss3, sq3, g3, bt3, cE)

    dst3 = dst.reshape(_NW, EPW // _K, _K)
    parts = _sc_scatter(E, N)(h3, dst3, jnp.concatenate([st3[0], st3[1]]))
    C = _tc_combine(parts, x, We1)
    prm = jnp.concatenate([be1, We2.reshape(-1), be2.reshape(-1),
                           jnp.zeros((15,), F32)])
    o = _sc_final(E, N)(C, src, dst, prm)
    return o.reshape(E, 1)


# speedup vs baseline: 1.4480x; 1.0205x over previous
import functools

import jax
import jax.numpy as jnp
from jax import lax
from jax.experimental import pallas as pl
from jax.experimental.pallas import tpu as pltpu
from jax.experimental.pallas import tpu_sc as plsc

F32 = jnp.float32
_NC = 2
_NS = 16
_NW = _NC * _NS
_L = 16
_H = 64
_D = 128
_K = 80


def _mesh():
    return plsc.VectorSubcoreMesh(core_axis_name="c", subcore_axis_name="s")


def _sc_gather(E, N):
    EPW = E // _NW
    NBLK = EPW // _K
    PAIRS = (NBLK - 1) // 2

    @functools.partial(
        pl.kernel,
        out_type=[
            jax.ShapeDtypeStruct((E, _H), F32),
            jax.ShapeDtypeStruct((_NW * _D,), F32),
        ],
        mesh=_mesh(),
        scratch_types=[
            pltpu.VMEM((EPW,), jnp.int32),
            pltpu.VMEM((EPW,), jnp.int32),
            pltpu.VMEM((_K, _D), F32),
            pltpu.VMEM((_K, _D), F32),
            pltpu.VMEM((_K, _D), F32),
            pltpu.VMEM((_K, _D), F32),
            pltpu.VMEM((_K, _H), F32),
            pltpu.VMEM((_K, _H), F32),
            pltpu.VMEM((_D,), F32),
            pltpu.SemaphoreType.DMA,
            pltpu.SemaphoreType.DMA,
            pltpu.SemaphoreType.DMA,
            pltpu.SemaphoreType.DMA,
        ],
    )
    def k(ab, srcv, dstv, h_out, st_out,
          isv, idv, ra0, ra1, rb0, rb1, hb0, hb1, sb, sA0, sA1, sO0, sO1):
        cid = lax.axis_index("c")
        sid = lax.axis_index("s")
        wid = sid * _NC + cid
        base = wid * EPW
        zero = jnp.zeros((_L,), F32)

        pltpu.sync_copy(srcv.at[pl.ds(base, EPW)], isv)
        pltpu.sync_copy(dstv.at[pl.ds(base, EPW)], idv)

        def issue(b, ra, rb, sem):
            pltpu.async_copy(ab.at[idv.at[pl.ds(b * _K, _K)]], ra, sem)
            pltpu.async_copy(ab.at[isv.at[pl.ds(b * _K, _K)]], rb, sem)

        def drain_g(ra, rb, sem):
            pltpu.make_async_copy(ab.at[pl.ds(0, _K)], ra, sem).wait()
            pltpu.make_async_copy(ab.at[pl.ds(0, _K)], rb, sem).wait()

        def out(b, hb, sem):
            pltpu.async_copy(hb, h_out.at[pl.ds(base + b * _K, _K)], sem)

        def drain_o(hb, sem):
            pltpu.make_async_copy(hb, h_out.at[pl.ds(base, _K)], sem).wait()

        def compute(ra, rb, hb, carry):
            def row(r, c8):
                acc = list(c8)
                for c in range(4):
                    h = (ra[r, pl.ds(c * _L, _L)]
                         + rb[r, pl.ds(_H + c * _L, _L)])
                    hb[r, pl.ds(c * _L, _L)] = h
                    acc[c] = acc[c] + h
                    acc[4 + c] = acc[4 + c] + h * h
                return tuple(acc)

            return lax.fori_loop(0, _K, row, carry, unroll=8)

        issue(0, ra0, rb0, sA0)
        out(0, hb0, sO0)
        out(1, hb1, sO1)

        def pair(t, carry):
            a = 2 * t
            issue(a + 1, ra1, rb1, sA1)
            drain_g(ra0, rb0, sA0)
            drain_o(hb0, sO0)
            carry = compute(ra0, rb0, hb0, carry)
            out(a, hb0, sO0)
            issue(a + 2, ra0, rb0, sA0)
            drain_g(ra1, rb1, sA1)
            drain_o(hb1, sO1)
            carry = compute(ra1, rb1, hb1, carry)
            out(a + 1, hb1, sO1)
            return carry

        carry = lax.fori_loop(0, PAIRS, pair, (zero,) * 8)
        drain_g(ra0, rb0, sA0)
        drain_o(hb0, sO0)
        carry = compute(ra0, rb0, hb0, carry)
        out(NBLK - 1, hb0, sO0)
        drain_o(hb0, sO0)
        drain_o(hb1, sO1)

        for c in range(4):
            sb[pl.ds(c * _L, _L)] = carry[c]
            sb[pl.ds(_H + c * _L, _L)] = carry[4 + c]
        pltpu.sync_copy(sb, st_out.at[pl.ds(wid * _D, _D)])

    return k


def _sc_scatter(E, N):
    EPW = E // _NW
    NBLK = EPW // _K
    PAIRS = (NBLK - 1) // 2
    CH = 40
    NCHUNK = N // CH
    TPW = (NCHUNK + _NS - 1) // _NS

    @functools.partial(
        pl.kernel,
        out_type=jax.ShapeDtypeStruct((_NC, N, _D), F32),
        mesh=_mesh(),
        scratch_types=[
            pltpu.VMEM_SHARED((N, _D), F32),
            pltpu.VMEM((NBLK, _K), jnp.int32),
            pltpu.VMEM((_K, _D), F32),
            pltpu.VMEM((_K, _D), F32),
            pltpu.VMEM((CH, _D), F32),
            pltpu.VMEM((_D,), F32),
            pltpu.SemaphoreType.DMA,
            pltpu.SemaphoreType.DMA,
        ],
    )
    def k(h3, dst3, st_hbm, part,
          acc, idx3, pb0, pb1, zbuf, stv, sS0, sS1):
        cid = lax.axis_index("c")
        sid = lax.axis_index("s")
        wid = sid * _NC + cid
        base = wid * EPW
        pltpu.sync_copy(st_hbm, stv)
        pltpu.sync_copy(dst3.at[wid], idx3)

        zv = jnp.zeros((_L,), F32)

        def zrow(r, _):
            for c in range(_D // _L):
                zbuf[r, pl.ds(c * _L, _L)] = zv
            return 0

        lax.fori_loop(0, CH, zrow, 0)
        for t in range(TPW):
            c = sid * TPW + t

            @pl.when(c < NCHUNK)
            def _():
                pltpu.sync_copy(zbuf, acc.at[pl.ds(c * CH, CH)])

        def pzrow(r, _):
            for c in range(_D // _L):
                pb0[r, pl.ds(c * _L, _L)] = zv
                pb1[r, pl.ds(c * _L, _L)] = zv
            return 0

        lax.fori_loop(0, _K, pzrow, 0)
        plsc.subcore_barrier()

        svec = [stv[pl.ds(c * _L, _L)] for c in range(4)]
        tvec = [stv[pl.ds(_H + c * _L, _L)] for c in range(4)]

        def scat(b, pb, sem):
            pltpu.async_copy(pb, acc.at[idx3.at[b]], sem, add=True)

        def drain_s(pb, sem):
            pltpu.make_async_copy(pb, acc.at[idx3.at[0]], sem).wait()

        def block(b, pb, sem):
            drain_s(pb, sem)
            pltpu.sync_copy(h3.at[pl.ds(base + b * _K, _K)], pb)

            def row(r, _2):
                for c in range(4):
                    v = pb[r, pl.ds(c * _L, _L)]
                    pb[r, pl.ds(c * _L, _L)] = jnp.maximum(
                        v * svec[c] + tvec[c], 0.0)
                return 0

            lax.fori_loop(0, _K, row, 0, unroll=8)
            scat(b, pb, sem)

        scat(0, pb0, sS0)
        scat(1, pb1, sS1)

        def pair(t, _):
            a = 2 * t
            block(a, pb0, sS0)
            block(a + 1, pb1, sS1)
            return 0

        lax.fori_loop(0, PAIRS, pair, 0)
        block(NBLK - 1, pb0, sS0)
        drain_s(pb0, sS0)
        drain_s(pb1, sS1)
        plsc.subcore_barrier()

        for t in range(TPW):
            c = sid * TPW + t

            @pl.when(c < NCHUNK)
            def _():
                pltpu.sync_copy(acc.at[pl.ds(c * CH, CH)], zbuf)
                pltpu.sync_copy(zbuf, part.at[cid, pl.ds(c * CH, CH)])

    return k


def _sc_final(E, N):
    EPW = E // _NW
    NBLK = EPW // _K
    PAIRS = (NBLK - 1) // 2

    @functools.partial(
        pl.kernel,
        out_type=jax.ShapeDtypeStruct((E,), F32),
        mesh=_mesh(),
        scratch_types=[
            pltpu.VMEM((EPW,), jnp.int32),
            pltpu.VMEM((EPW,), jnp.int32),
            pltpu.VMEM((_K, _D), F32),
            pltpu.VMEM((_K, _D), F32),
            pltpu.VMEM((_K, _D), F32),
            pltpu.VMEM((_K, _D), F32),
            pltpu.VMEM((_K + _L,), F32),
            pltpu.VMEM((_K + _L,), F32),
            pltpu.VMEM((272,), F32),
            pltpu.SemaphoreType.DMA,
            pltpu.SemaphoreType.DMA,
            pltpu.SemaphoreType.DMA,
            pltpu.SemaphoreType.DMA,
        ],
    )
    def k(c_hbm, srcv, dstv, prm, o_hbm,
          isv, idv, rs0, rs1, rd0, rd1, ob0, ob1, pv, sA0, sA1, sO0, sO1):
        cid = lax.axis_index("c")
        sid = lax.axis_index("s")
        wid = sid * _NC + cid
        base = wid * EPW

        pltpu.sync_copy(prm, pv)
        pltpu.sync_copy(srcv.at[pl.ds(base, EPW)], isv)
        pltpu.sync_copy(dstv.at[pl.ds(base, EPW)], idv)
        lanes = lax.iota(jnp.int32, _L)
        bev = [pv[pl.ds(c * _L, _L)] for c in range(_D // _L)]
        wev = [pv[pl.ds(_D + c * _L, _L)] for c in range(_D // _L)]
        be2s = pv[pl.ds(2 * _D, _L)][0]

        def issue(b, rs, rd, sem):
            pltpu.async_copy(c_hbm.at[isv.at[pl.ds(b * _K, _K)]], rs, sem)
            pltpu.async_copy(c_hbm.at[idv.at[pl.ds(b * _K, _K)]], rd, sem)

        def drain_g(rs, rd, sem):
            pltpu.make_async_copy(c_hbm.at[pl.ds(0, _K)], rs, sem).wait()
            pltpu.make_async_copy(c_hbm.at[pl.ds(0, _K)], rd, sem).wait()

        def out(b, ob, sem):
            pltpu.async_copy(ob.at[pl.ds(0, _K)],
                             o_hbm.at[pl.ds(base + b * _K, _K)], sem)

        def drain_o(ob, sem):
            pltpu.make_async_copy(ob.at[pl.ds(0, _K)],
                                  o_hbm.at[pl.ds(base, _K)], sem).wait()

        def compute(rs, rd, ob):
            def row(r, ovec):
                accs = []
                for c in range(_D // _L):
                    dv = rs[r, pl.ds(c * _L, _L)] - rd[r, pl.ds(c * _L, _L)]
                    dv = jnp.maximum(dv + bev[c], 0.0)
                    accs.append(dv * wev[c])
                while len(accs) > 1:
                    accs = [a + b for a, b in zip(accs[::2], accs[1::2])]
                parts = [accs[0][l] for l in range(_L)]
                while len(parts) > 1:
                    parts = [p + q for p, q in zip(parts[::2], parts[1::2])]
                z = parts[0] + be2s
                lr = lax.rem(r, _L)
                ovec = jnp.where(lanes == lr, z, ovec)

                @pl.when((lr == _L - 1) | (r == _K - 1))
                def _():
                    ob[pl.ds(r - lr, _L)] = 1.0 / (1.0 + jnp.exp(-ovec))

                return ovec

            lax.fori_loop(0, _K, row, jnp.zeros((_L,), F32), unroll=8)

        issue(0, rs0, rd0, sA0)
        out(0, ob0, sO0)
        out(1, ob1, sO1)

        def pair(t, _):
            a = 2 * t
            issue(a + 1, rs1, rd1, sA1)
            drain_g(rs0, rd0, sA0)
            drain_o(ob0, sO0)
            compute(rs0, rd0, ob0)
            out(a, ob0, sO0)
            issue(a + 2, rs0, rd0, sA0)
            drain_g(rs1, rd1, sA1)
            drain_o(ob1, sO1)
            compute(rs1, rd1, ob1)
            out(a + 1, ob1, sO1)
            return 0

        lax.fori_loop(0, PAIRS, pair, 0)
        drain_g(rs0, rd0, sA0)
        drain_o(ob0, sO0)
        compute(rs0, rd0, ob0)
        out(NBLK - 1, ob0, sO0)
        drain_o(ob0, sO0)
        drain_o(ob1, sO1)

    return k


def _tc_pre(x, W1):
    N, D = x.shape
    BN = 5000

    def body(x_ref, w_ref, ab_ref):
        w = w_ref[...]
        wa = w[:D, :]
        wb = w[D:, :]
        xb = x_ref[...]
        ab_ref[...] = jnp.concatenate(
            [jnp.dot(xb, wa - wb, preferred_element_type=F32),
             jnp.dot(xb, wb, preferred_element_type=F32)], axis=1)

    return pl.pallas_call(
        body,
        grid=(N // BN,),
        in_specs=[pl.BlockSpec((BN, D), lambda i: (i, 0)),
                  pl.BlockSpec((2 * D, _H), lambda i: (0, 0))],
        out_specs=pl.BlockSpec((BN, 2 * _H), lambda i: (i, 0)),
        out_shape=jax.ShapeDtypeStruct((N, 2 * _H), F32),
    )(x, W1)


def _tc_mid(h, st, W, pad_count=False):
    E = h.shape[0]
    BE = 16000
    G = E // BE
    OW = _D if pad_count else _H

    def body(h_ref, st_ref, w_ref, o_ref, ss_ref, sq_ref):
        i = pl.program_id(0)
        stb = st_ref[...]
        hb = h_ref[...].astype(F32)
        h1 = jnp.maximum(hb * stb[0:1, :] + stb[1:2, :], 0.0)
        o = jnp.dot(h1, w_ref[...], preferred_element_type=F32)
        if pad_count:
            cols = lax.broadcasted_iota(jnp.int32, (BE, _H), 1)
            cc = jnp.where(cols == 0, jnp.float32(1), jnp.float32(0))
            o_ref[...] = jnp.concatenate([o, cc], axis=1)
        else:
            o_ref[...] = o.astype(jnp.bfloat16)
        s = jnp.broadcast_to(jnp.sum(o, axis=0, keepdims=True), (8, _H))
        q = jnp.broadcast_to(jnp.sum(o * o, axis=0, keepdims=True), (8, _H))

        @pl.when(i == 0)
        def _():
            ss_ref[...] = s
            sq_ref[...] = q

        @pl.when(i > 0)
        def _():
            ss_ref[...] = ss_ref[...] + s
            sq_ref[...] = sq_ref[...] + q

    out, ss, sq = pl.pallas_call(
        body,
        grid=(G,),
        in_specs=[pl.BlockSpec((BE, _H), lambda i: (i, 0)),
                  pl.BlockSpec((2, _H), lambda i: (0, 0)),
                  pl.BlockSpec((_H, _H), lambda i: (0, 0))],
        out_specs=[pl.BlockSpec((BE, OW), lambda i: (i, 0)),
                   pl.BlockSpec((8, _H), lambda i: (0, 0)),
                   pl.BlockSpec((8, _H), lambda i: (0, 0))],
        out_shape=[jax.ShapeDtypeStruct((E, OW),
                                        F32 if pad_count else jnp.bfloat16),
                   jax.ShapeDtypeStruct((8, _H), F32),
                   jax.ShapeDtypeStruct((8, _H), F32)],
    )(h, st, W)
    return out, ss[0], sq[0]


def _tc_combine(parts, x, We1):
    N, D = x.shape
    BN = 5000

    def body(p_ref, x_ref, w_ref, c_ref):
        p = p_ref[...]
        ps = p[0] + p[1]
        cnt = jnp.maximum(ps[:, _H:_H + 1], 1.0)
        agg = jnp.maximum(ps[:, :_H] / cnt, 0.0)
        xb = jnp.maximum(x_ref[...], 0.0)
        w = w_ref[...]
        c_ref[...] = (jnp.dot(agg, w[:_H, :], preferred_element_type=F32)
                      + jnp.dot(xb, w[_H:, :], preferred_element_type=F32))

    return pl.pallas_call(
        body,
        grid=(N // BN,),
        in_specs=[pl.BlockSpec((2, BN, _D), lambda i: (0, i, 0)),
                  pl.BlockSpec((BN, D), lambda i: (i, 0)),
                  pl.BlockSpec((_H + D, D), lambda i: (0, 0))],
        out_specs=pl.BlockSpec((BN, D), lambda i: (i, 0)),
        out_shape=jax.ShapeDtypeStruct((N, D), F32),
    )(parts, x, We1)


def _affine(ssum, ssq, g, bt, count):
    m = ssum / count
    v = ssq / count - m * m
    s = g * lax.rsqrt(v + 1e-5)
    t = bt - m * s
    return jnp.stack([s, t])


def kernel(x, edge_index, W1, b1, g1, bt1, W2, b2, g2, bt2,
           W3, b3, g3, bt3, We1, be1, We2, be2):
    N, D = x.shape
    E = edge_index.shape[1]
    EPW = E // _NW
    src = edge_index[0]
    dst = edge_index[1]

    AB = _tc_pre(x, W1)
    h1, st1f = _sc_gather(E, N)(AB, src, dst)
    st1p = st1f.reshape(_NW, _D)
    cE = jnp.float32(E)
    st1 = _affine(jnp.sum(st1p[:, :_H], axis=0),
                  jnp.sum(st1p[:, _H:], axis=0), g1, bt1, cE)
    h2, ss2, sq2 = _tc_mid(h1, st1, W2)
    st2 = _affine(ss2, sq2, g2, bt2, cE)
    h3, ss3, sq3 = _tc_mid(h2, st2, W3, pad_count=True)
    st3 = _affine(ss3, sq3, g3, bt3, cE)

    dst3 = dst.reshape(_NW, EPW // _K, _K)
    parts = _sc_scatter(E, N)(h3, dst3, jnp.concatenate([st3[0], st3[1]]))
    C = _tc_combine(parts, x, We1)
    prm = jnp.concatenate([be1, We2.reshape(-1), be2.reshape(-1),
                           jnp.zeros((15,), F32)])
    o = _sc_final(E, N)(C, src, dst, prm)
    return o.reshape(E, 1)
